# baseline (speedup 1.0000x reference)
import jax
import jax.numpy as jnp
from jax import lax
from jax.experimental import pallas as pl
from jax.experimental.pallas import tpu as pltpu
from jax.experimental.pallas import tpu_sc as plsc

VOCAB = 1000000
EMB = 64
SEQ = 200
BATCH = 1024

NC = 2
NS = 16
L = 16
NW = NC * NS
BPW = BATCH // NW
BPC = 2
NCHUNK = BPW // BPC
CHUNK = BPC * SEQ
G = 100


def _emb_body(idx_hbm, table_hbm, pos_hbm, out_hbm, idx_v, rows_v, out_v, pos_v, sem):
    wid = lax.axis_index("s") * NC + lax.axis_index("c")
    pltpu.sync_copy(pos_hbm, pos_v)
    for c in range(NCHUNK):
        b0 = pl.multiple_of(wid * BPW + c * BPC, BPC)
        pltpu.sync_copy(idx_hbm.at[pl.ds(b0, BPC)], idx_v)
        copies = [
            pltpu.async_copy(
                table_hbm.at[idx_v.at[b, k]],
                rows_v.at[pl.ds((b * SEQ + k * G), G)],
                sem,
            )
            for b in range(BPC)
            for k in range(SEQ // G)
        ]
        for cp in copies:
            cp.wait()

        def body(t, carry):
            for j in range(EMB // L):
                p = pos_v[t, pl.ds(j * L, L)]
                for b in range(BPC):
                    r = b * SEQ + t
                    out_v[b, t, pl.ds(j * L, L)] = rows_v[r, pl.ds(j * L, L)] + p
            return carry

        lax.fori_loop(0, SEQ, body, 0)
        pltpu.sync_copy(out_v, out_hbm.at[pl.ds(b0, BPC)])


def kernel(sentence, word_table, pos_table):
    wt128 = jnp.pad(word_table, ((0, 0), (0, 128 - EMB)))
    idx = jnp.transpose(sentence, (1, 0)).reshape(BATCH, SEQ // G, G)
    pos = lax.slice_in_dim(pos_table, 1, SEQ + 1, axis=0)
    mesh = plsc.VectorSubcoreMesh(core_axis_name="c", subcore_axis_name="s")
    out = pl.kernel(
        _emb_body,
        out_type=jax.ShapeDtypeStruct((BATCH, SEQ, EMB), jnp.float32),
        mesh=mesh,
        compiler_params=pltpu.CompilerParams(use_tc_tiling_on_sc=False),
        scratch_types=[
            pltpu.VMEM((BPC, SEQ // G, G), jnp.int32),
            pltpu.VMEM((CHUNK, 128), jnp.float32),
            pltpu.VMEM((BPC, SEQ, EMB), jnp.float32),
            pltpu.VMEM((SEQ, EMB), jnp.float32),
            pltpu.SemaphoreType.DMA,
        ],
    )(idx, wt128, pos)
    return out
